# baseline (speedup 1.0000x reference)
import functools

import jax
import jax.numpy as jnp
from jax import lax
from jax.experimental import pallas as pl
from jax.experimental.pallas import tpu as pltpu
from jax.experimental.pallas import tpu_sc as plsc

VOCAB = 8192
DIM = 64
BETA = 0.25
M_TOTAL = 4096
K_BLK = 2048
M_BLK = 4096
NK = VOCAB // K_BLK
NM = M_TOTAL // M_BLK

SC_CORES = 2
SC_SUBCORES = 16
SC_WORKERS = SC_CORES * SC_SUBCORES
ROWS_PER_WORKER = M_TOTAL // SC_WORKERS


def _vq_tc_body(x_ref, dict_ref, counts_ref, xsq_ref, esq_ref, counts_r_ref,
                e_ref, ids_ref, loss_ref, perp_ref,
                best_ref, bidx_ref):
    k = pl.program_id(0)
    m = pl.program_id(1)
    x = x_ref[...]
    d = dict_ref[...]
    c = counts_ref[...]
    e = d / c

    @pl.when(m == 0)
    def _():
        e_ref[...] = jnp.concatenate([e, jnp.zeros_like(e)], axis=1)

    dots2 = lax.dot_general(-2.0 * e, x, (((1,), (1,)), ((), ())))
    scores = (xsq_ref[...] + dots2) + esq_ref[...]

    bmin = jnp.min(scores, axis=0, keepdims=True)
    iota = lax.broadcasted_iota(jnp.int32, scores.shape, 0)
    hit = jnp.where(scores == bmin, iota, VOCAB)
    bidx = jnp.min(hit, axis=0, keepdims=True) + k * K_BLK

    msl = pl.ds(m, 1)

    @pl.when(k == 0)
    def _():
        best_ref[msl, :] = bmin
        bidx_ref[msl, :] = bidx

    @pl.when(k > 0)
    def _():
        prev = best_ref[msl, :]
        pidx = bidx_ref[msl, :]
        better = bmin < prev
        best_ref[msl, :] = jnp.where(better, bmin, prev)
        bidx_ref[msl, :] = jnp.where(better, bidx, pidx)

    @pl.when(k == NK - 1)
    def _():
        ids_ref[...] = bidx_ref[msl, :]

        @pl.when(m == NM - 1)
        def _():
            e_latent = jnp.sum(best_ref[...]) / (M_TOTAL * DIM)
            loss_ref[...] = (BETA * e_latent).reshape(1, 1)
            cf = counts_r_ref[...]
            s = jnp.sum(cf)
            p = cf / s
            perp_ref[...] = jnp.exp(-jnp.sum(p * jnp.log(p + 1e-10))).reshape(1, 1)


def _vq_argmin_tc(x, dict2, counts2, xsq, esq, counts_r):
    return pl.pallas_call(
        _vq_tc_body,
        grid=(NK, NM),
        in_specs=[
            pl.BlockSpec((M_BLK, DIM), lambda k, m: (m, 0)),
            pl.BlockSpec((K_BLK, DIM), lambda k, m: (k, 0)),
            pl.BlockSpec((K_BLK, 1), lambda k, m: (k, 0)),
            pl.BlockSpec((1, M_BLK), lambda k, m: (0, m)),
            pl.BlockSpec((K_BLK, 1), lambda k, m: (k, 0)),
            pl.BlockSpec((64, 128), lambda k, m: (0, 0)),
        ],
        out_specs=[
            pl.BlockSpec((K_BLK, 2 * DIM), lambda k, m: (k, 0)),
            pl.BlockSpec((1, M_BLK), lambda k, m: (0, m)),
            pl.BlockSpec((1, 1), lambda k, m: (0, 0)),
            pl.BlockSpec((1, 1), lambda k, m: (0, 0)),
        ],
        out_shape=[
            jax.ShapeDtypeStruct((VOCAB, 2 * DIM), jnp.float32),
            jax.ShapeDtypeStruct((1, M_TOTAL), jnp.int32),
            jax.ShapeDtypeStruct((1, 1), jnp.float32),
            jax.ShapeDtypeStruct((1, 1), jnp.float32),
        ],
        scratch_shapes=[
            pltpu.VMEM((NM, M_BLK), jnp.float32),
            pltpu.VMEM((NM, M_BLK), jnp.int32),
        ],
        compiler_params=pltpu.CompilerParams(
            dimension_semantics=("arbitrary", "arbitrary"),
        ),
    )(x, dict2, counts2, xsq, esq, counts_r)


def _sc_gather_body(e_hbm, ids_hbm, out_hbm, idx_v, rows_v, sem):
    wid = lax.axis_index("s") * SC_CORES + lax.axis_index("c")
    base = wid * ROWS_PER_WORKER
    pltpu.sync_copy(ids_hbm.at[pl.ds(base, ROWS_PER_WORKER)], idx_v)
    pltpu.async_copy(e_hbm.at[idx_v], rows_v, sem).wait()
    pltpu.sync_copy(rows_v, out_hbm.at[pl.ds(base, ROWS_PER_WORKER)])


def _sc_gather(e, ids):
    fn = functools.partial(
        pl.kernel,
        out_type=jax.ShapeDtypeStruct((M_TOTAL, 2 * DIM), jnp.float32),
        mesh=plsc.VectorSubcoreMesh(core_axis_name="c", subcore_axis_name="s"),
        scratch_types=[
            pltpu.VMEM((ROWS_PER_WORKER,), jnp.int32),
            pltpu.VMEM((ROWS_PER_WORKER, 2 * DIM), jnp.float32),
            pltpu.SemaphoreType.DMA,
        ],
    )(_sc_gather_body)
    return fn(e, ids)


def kernel(inputs, dictionary, counts):
    input_shape = inputs.shape
    x = inputs.reshape(M_TOTAL, DIM)
    counts2 = counts.reshape(VOCAB, 1)
    counts_r = counts.reshape(64, 128)
    xsq = (x ** 2).sum(axis=1, keepdims=True).reshape(1, M_TOTAL)
    esq = ((dictionary / counts[:, None]) ** 2).sum(axis=1).reshape(VOCAB, 1)
    e, ids, loss, perp = _vq_argmin_tc(x, dictionary, counts2, xsq, esq, counts_r)
    x_q = _sc_gather(e, ids.reshape(M_TOTAL))[:, :DIM]
    return (x_q.reshape(input_shape), loss[0, 0], perp[0, 0])


# speedup vs baseline: 1.0038x; 1.0038x over previous
import functools

import jax
import jax.numpy as jnp
from jax import lax
from jax.experimental import pallas as pl
from jax.experimental.pallas import tpu as pltpu
from jax.experimental.pallas import tpu_sc as plsc

VOCAB = 8192
DIM = 64
BETA = 0.25
M_TOTAL = 4096
K_BLK = 1024
M_BLK = 4096
NK = VOCAB // K_BLK
NM = M_TOTAL // M_BLK

SC_CORES = 2
SC_SUBCORES = 16
SC_WORKERS = SC_CORES * SC_SUBCORES
ROWS_PER_WORKER = M_TOTAL // SC_WORKERS


def _vq_tc_body(x_ref, dict_ref, counts_ref, xsq_ref, esq_ref, counts_r_ref,
                e_ref, ids_ref, loss_ref, perp_ref,
                best_ref, bidx_ref):
    k = pl.program_id(0)
    m = pl.program_id(1)
    x = x_ref[...]
    d = dict_ref[...]
    c = counts_ref[...]
    e = d / c

    @pl.when(m == 0)
    def _():
        e_ref[...] = jnp.concatenate([e, jnp.zeros_like(e)], axis=1)

    dots2 = lax.dot_general(-2.0 * e, x, (((1,), (1,)), ((), ())))
    scores = (xsq_ref[...] + dots2) + esq_ref[...]

    bmin = jnp.min(scores, axis=0, keepdims=True)
    iota = lax.broadcasted_iota(jnp.int32, scores.shape, 0)
    hit = jnp.where(scores == bmin, iota, VOCAB)
    bidx = jnp.min(hit, axis=0, keepdims=True) + k * K_BLK

    msl = pl.ds(m, 1)

    @pl.when(k == 0)
    def _():
        best_ref[msl, :] = bmin
        bidx_ref[msl, :] = bidx

    @pl.when(k > 0)
    def _():
        prev = best_ref[msl, :]
        pidx = bidx_ref[msl, :]
        better = bmin < prev
        best_ref[msl, :] = jnp.where(better, bmin, prev)
        bidx_ref[msl, :] = jnp.where(better, bidx, pidx)

    @pl.when(k == NK - 1)
    def _():
        ids_ref[...] = bidx_ref[msl, :]

        @pl.when(m == NM - 1)
        def _():
            e_latent = jnp.sum(best_ref[...]) / (M_TOTAL * DIM)
            loss_ref[...] = (BETA * e_latent).reshape(1, 1)
            cf = counts_r_ref[...]
            s = jnp.sum(cf)
            p = cf / s
            perp_ref[...] = jnp.exp(-jnp.sum(p * jnp.log(p + 1e-10))).reshape(1, 1)


def _vq_argmin_tc(x, dict2, counts2, xsq, esq, counts_r):
    return pl.pallas_call(
        _vq_tc_body,
        grid=(NK, NM),
        in_specs=[
            pl.BlockSpec((M_BLK, DIM), lambda k, m: (m, 0)),
            pl.BlockSpec((K_BLK, DIM), lambda k, m: (k, 0)),
            pl.BlockSpec((K_BLK, 1), lambda k, m: (k, 0)),
            pl.BlockSpec((1, M_BLK), lambda k, m: (0, m)),
            pl.BlockSpec((K_BLK, 1), lambda k, m: (k, 0)),
            pl.BlockSpec((64, 128), lambda k, m: (0, 0)),
        ],
        out_specs=[
            pl.BlockSpec((K_BLK, 2 * DIM), lambda k, m: (k, 0)),
            pl.BlockSpec((1, M_BLK), lambda k, m: (0, m)),
            pl.BlockSpec((1, 1), lambda k, m: (0, 0)),
            pl.BlockSpec((1, 1), lambda k, m: (0, 0)),
        ],
        out_shape=[
            jax.ShapeDtypeStruct((VOCAB, 2 * DIM), jnp.float32),
            jax.ShapeDtypeStruct((1, M_TOTAL), jnp.int32),
            jax.ShapeDtypeStruct((1, 1), jnp.float32),
            jax.ShapeDtypeStruct((1, 1), jnp.float32),
        ],
        scratch_shapes=[
            pltpu.VMEM((NM, M_BLK), jnp.float32),
            pltpu.VMEM((NM, M_BLK), jnp.int32),
        ],
        compiler_params=pltpu.CompilerParams(
            dimension_semantics=("arbitrary", "arbitrary"),
        ),
    )(x, dict2, counts2, xsq, esq, counts_r)


def _sc_gather_body(e_hbm, ids_hbm, out_hbm, idx_v, rows_v, sem):
    wid = lax.axis_index("s") * SC_CORES + lax.axis_index("c")
    base = wid * ROWS_PER_WORKER
    pltpu.sync_copy(ids_hbm.at[pl.ds(base, ROWS_PER_WORKER)], idx_v)
    pltpu.async_copy(e_hbm.at[idx_v], rows_v, sem).wait()
    pltpu.sync_copy(rows_v, out_hbm.at[pl.ds(base, ROWS_PER_WORKER)])


def _sc_gather(e, ids):
    fn = functools.partial(
        pl.kernel,
        out_type=jax.ShapeDtypeStruct((M_TOTAL, 2 * DIM), jnp.float32),
        mesh=plsc.VectorSubcoreMesh(core_axis_name="c", subcore_axis_name="s"),
        scratch_types=[
            pltpu.VMEM((ROWS_PER_WORKER,), jnp.int32),
            pltpu.VMEM((ROWS_PER_WORKER, 2 * DIM), jnp.float32),
            pltpu.SemaphoreType.DMA,
        ],
    )(_sc_gather_body)
    return fn(e, ids)


def kernel(inputs, dictionary, counts):
    input_shape = inputs.shape
    x = inputs.reshape(M_TOTAL, DIM)
    counts2 = counts.reshape(VOCAB, 1)
    counts_r = counts.reshape(64, 128)
    xsq = (x ** 2).sum(axis=1, keepdims=True).reshape(1, M_TOTAL)
    esq = ((dictionary / counts[:, None]) ** 2).sum(axis=1).reshape(VOCAB, 1)
    e, ids, loss, perp = _vq_argmin_tc(x, dictionary, counts2, xsq, esq, counts_r)
    x_q = _sc_gather(e, ids.reshape(M_TOTAL))[:, :DIM]
    return (x_q.reshape(input_shape), loss[0, 0], perp[0, 0])
